# baseline (speedup 1.0000x reference)
import functools

import jax
import jax.numpy as jnp
from jax import lax
from jax.experimental import pallas as pl
from jax.experimental.pallas import tpu as pltpu
from jax.experimental.pallas import tpu_sc as plsc

_LANES = 16
_SKILL_PAD = 1024
_PACK_SHIFT = 12


@functools.lru_cache(maxsize=None)
def _make_sc_stats(B, T):
    NC, NS = 2, 16
    NW = NC * NS
    rows_per_w = B // NW
    n_groups = rows_per_w // _LANES
    mesh = plsc.VectorSubcoreMesh(core_axis_name="c", subcore_axis_name="s")
    blk = _LANES * T

    wblk = n_groups * blk

    @functools.partial(
        pl.kernel,
        mesh=mesh,
        compiler_params=pltpu.CompilerParams(needs_layout_passes=False),
        out_type=[jax.ShapeDtypeStruct((B * T,), jnp.float32)] * 2,
        scratch_types=[
            pltpu.VMEM((n_groups * _LANES * _SKILL_PAD,), jnp.int32),
            pltpu.VMEM((wblk,), jnp.int32),
            pltpu.VMEM((wblk,), jnp.int32),
            pltpu.VMEM((wblk,), jnp.float32),
            pltpu.VMEM((wblk,), jnp.float32),
        ],
    )
    def stats(q_hbm, r_hbm, ztbl_hbm, xa_hbm, xm_hbm,
              tbl, qv, rv, xa, xm):
        wid = lax.axis_index("s") * NC + lax.axis_index("c")
        lanes = lax.iota(jnp.int32, _LANES)
        lane_t = lanes * T
        lane_tbl = lanes * _SKILL_PAD
        base = wid * wblk
        pltpu.sync_copy(ztbl_hbm, tbl)
        pltpu.sync_copy(q_hbm.at[pl.ds(base, wblk)], qv)
        pltpu.sync_copy(r_hbm.at[pl.ds(base, wblk)], rv)

        def body(t, carry):
            for g in range(n_groups):
                pos = lane_t + (g * blk + t)
                qt = plsc.load_gather(qv, [pos])
                rt = plsc.load_gather(rv, [pos])
                adr = lane_tbl + (g * _LANES * _SKILL_PAD) + qt
                v = plsc.load_gather(tbl, [adr])
                v2 = v + ((1 << _PACK_SHIFT) + rt)
                plsc.store_scatter(tbl, [adr], v2)
                att = v2 >> _PACK_SHIFT
                cor = v2 & ((1 << _PACK_SHIFT) - 1)
                af = att.astype(jnp.float32)
                mf = cor.astype(jnp.float32) / jnp.maximum(af, 1.0)
                plsc.store_scatter(xa, [pos], af)
                plsc.store_scatter(xm, [pos], mf)
            return carry

        lax.fori_loop(0, T, body, 0)
        pltpu.sync_copy(xa, xa_hbm.at[pl.ds(base, wblk)])
        pltpu.sync_copy(xm, xm_hbm.at[pl.ds(base, wblk)])

    return stats


def _tc_expand_body(xs_ref, xa_ref, xm_ref, e_ref, wp_ref, bp_ref, *rest):
    o_ref = rest[-1]
    g = lax.dot_general(e_ref[...], wp_ref[...], (((0,), (1,)), ((), ())),
                        precision=lax.Precision.HIGHEST,
                        preferred_element_type=jnp.float32)
    gb = jnp.concatenate([g[0:3, :], g[3:4, :] + bp_ref[...]], axis=0)
    ones = jnp.ones(xs_ref.shape, jnp.float32)
    x = jnp.concatenate([xs_ref[...], xa_ref[...], xm_ref[...], ones], axis=0)
    o_ref[...] = lax.dot_general(x, gb, (((0,), (0,)), ((), ())),
                                 preferred_element_type=jnp.float32)


@functools.lru_cache(maxsize=None)
def _make_tc_expand(M, CD, EMB, Rm, nblk, blk0, alias):
    in_specs = [
        pl.BlockSpec((1, Rm), lambda i: (0, i)),
        pl.BlockSpec((1, Rm), lambda i: (0, i)),
        pl.BlockSpec((1, Rm), lambda i: (0, i)),
        pl.BlockSpec((CD, 4), lambda i: (0, 0)),
        pl.BlockSpec((EMB, CD), lambda i: (0, 0)),
        pl.BlockSpec((1, EMB), lambda i: (0, 0)),
    ]
    kwargs = {}
    if alias:
        in_specs.append(pl.BlockSpec(memory_space=pltpu.MemorySpace.HBM))
        kwargs["input_output_aliases"] = {6: 0}
    return pl.pallas_call(
        _tc_expand_body,
        grid=(nblk,),
        in_specs=in_specs,
        out_specs=pl.BlockSpec((Rm, EMB), lambda i: (blk0 + i, 0)),
        out_shape=jax.ShapeDtypeStruct((M, EMB), jnp.float32),
        **kwargs,
    )


def kernel(q, r, qry, Ws, bs, Wn, bn, Wm, bm, Wp, bp):
    B, T = q.shape
    EMB, CD = Wp.shape
    o1 = CD // 3
    f32 = jnp.float32
    q = q.astype(jnp.int32)
    r = r.astype(jnp.int32)

    M = B * T
    n_groups = (B // 32) // _LANES
    ztbl = jnp.zeros((n_groups * _LANES * _SKILL_PAD,), jnp.int32)
    xa, xm = _make_sc_stats(B, T)(q.reshape(M), r.reshape(M), ztbl)
    xs = q.reshape(M).astype(f32)

    z = lambda n: jnp.zeros((n,), f32)
    col0 = jnp.concatenate([Ws[:, 0], z(CD - o1)])
    col1 = jnp.concatenate([z(o1), Wn[:, 0], z(CD - 2 * o1)])
    col2 = jnp.concatenate([z(2 * o1), Wm[:, 0]])
    col3 = jnp.concatenate([bs, bn, bm])
    e_mat = jnp.stack([col0, col1, col2, col3], axis=1)

    Rm = 40960
    out = _make_tc_expand(M, CD, EMB, Rm, M // Rm, 0, False)(
        xs.reshape(1, M), xa.reshape(1, M), xm.reshape(1, M),
        e_mat, Wp, bp.reshape(1, EMB))
    return out.reshape(B, T, EMB)


# speedup vs baseline: 1.0166x; 1.0166x over previous
import functools

import jax
import jax.numpy as jnp
from jax import lax
from jax.experimental import pallas as pl
from jax.experimental.pallas import tpu as pltpu
from jax.experimental.pallas import tpu_sc as plsc

_LANES = 16
_SKILL_PAD = 1024
_PACK_SHIFT = 12


@functools.lru_cache(maxsize=None)
def _make_sc_stats(B, T):
    NC, NS = 2, 16
    NW = NC * NS
    rows_per_w = B // NW
    n_groups = rows_per_w // _LANES
    mesh = plsc.VectorSubcoreMesh(core_axis_name="c", subcore_axis_name="s")
    blk = _LANES * T

    wblk = n_groups * blk

    @functools.partial(
        pl.kernel,
        mesh=mesh,
        compiler_params=pltpu.CompilerParams(needs_layout_passes=False),
        out_type=[jax.ShapeDtypeStruct((B * T,), jnp.float32)] * 2,
        scratch_types=[
            pltpu.VMEM((n_groups * _LANES * _SKILL_PAD,), jnp.int32),
            pltpu.VMEM((wblk,), jnp.int32),
            pltpu.VMEM((wblk,), jnp.int32),
            pltpu.VMEM((wblk,), jnp.float32),
            pltpu.VMEM((wblk,), jnp.float32),
        ],
    )
    def stats(q_hbm, r_hbm, ztbl_hbm, xa_hbm, xm_hbm,
              tbl, qv, rv, xa, xm):
        wid = lax.axis_index("s") * NC + lax.axis_index("c")
        lanes = lax.iota(jnp.int32, _LANES)
        lane_t = lanes * T
        lane_tbl = lanes * _SKILL_PAD
        base = wid * wblk
        pltpu.sync_copy(ztbl_hbm, tbl)
        pltpu.sync_copy(q_hbm.at[pl.ds(base, wblk)], qv)
        pltpu.sync_copy(r_hbm.at[pl.ds(base, wblk)], rv)

        def body(t, carry):
            for g in range(n_groups):
                pos = lane_t + (g * blk + t)
                qt = plsc.load_gather(qv, [pos])
                rt = plsc.load_gather(rv, [pos])
                adr = lane_tbl + (g * _LANES * _SKILL_PAD) + qt
                v = plsc.load_gather(tbl, [adr])
                v2 = v + ((1 << _PACK_SHIFT) + rt)
                plsc.store_scatter(tbl, [adr], v2)
                att = v2 >> _PACK_SHIFT
                cor = v2 & ((1 << _PACK_SHIFT) - 1)
                af = att.astype(jnp.float32)
                mf = cor.astype(jnp.float32) / jnp.maximum(af, 1.0)
                plsc.store_scatter(xa, [pos], af)
                plsc.store_scatter(xm, [pos], mf)
            return carry

        lax.fori_loop(0, T, body, 0)
        pltpu.sync_copy(xa, xa_hbm.at[pl.ds(base, wblk)])
        pltpu.sync_copy(xm, xm_hbm.at[pl.ds(base, wblk)])

    return stats


def _tc_expand_body(xs_ref, xa_ref, xm_ref, e_ref, wp_ref, bp_ref, *rest):
    o_ref = rest[-1]
    g = lax.dot_general(e_ref[...], wp_ref[...], (((0,), (1,)), ((), ())),
                        precision=lax.Precision.HIGHEST,
                        preferred_element_type=jnp.float32)
    gb = jnp.concatenate([g[0:3, :], g[3:4, :] + bp_ref[...]], axis=0)
    ones = jnp.ones(xs_ref.shape, jnp.float32)
    x = jnp.concatenate([xs_ref[...], xa_ref[...], xm_ref[...], ones], axis=0)
    o_ref[...] = lax.dot_general(x, gb, (((0,), (0,)), ((), ())),
                                 preferred_element_type=jnp.float32)


@functools.lru_cache(maxsize=None)
def _make_tc_expand(M, CD, EMB, Rm, nblk, blk0, alias):
    in_specs = [
        pl.BlockSpec((1, Rm), lambda i: (0, i)),
        pl.BlockSpec((1, Rm), lambda i: (0, i)),
        pl.BlockSpec((1, Rm), lambda i: (0, i)),
        pl.BlockSpec((CD, 4), lambda i: (0, 0)),
        pl.BlockSpec((EMB, CD), lambda i: (0, 0)),
        pl.BlockSpec((1, EMB), lambda i: (0, 0)),
    ]
    kwargs = {}
    if alias:
        in_specs.append(pl.BlockSpec(memory_space=pltpu.MemorySpace.HBM))
        kwargs["input_output_aliases"] = {6: 0}
    return pl.pallas_call(
        _tc_expand_body,
        grid=(nblk,),
        in_specs=in_specs,
        out_specs=pl.BlockSpec((Rm, EMB), lambda i: (blk0 + i, 0)),
        out_shape=jax.ShapeDtypeStruct((M, EMB), jnp.float32),
        **kwargs,
    )


def kernel(q, r, qry, Ws, bs, Wn, bn, Wm, bm, Wp, bp):
    B, T = q.shape
    EMB, CD = Wp.shape
    o1 = CD // 3
    f32 = jnp.float32
    q = q.astype(jnp.int32)
    r = r.astype(jnp.int32)

    M = B * T
    n_groups = (B // 32) // _LANES
    ztbl = jnp.zeros((n_groups * _LANES * _SKILL_PAD,), jnp.int32)
    xa, xm = _make_sc_stats(B, T)(q.reshape(M), r.reshape(M), ztbl)
    xs = q.reshape(M).astype(f32)

    z = lambda n: jnp.zeros((n,), f32)
    col0 = jnp.concatenate([Ws[:, 0], z(CD - o1)])
    col1 = jnp.concatenate([z(o1), Wn[:, 0], z(CD - 2 * o1)])
    col2 = jnp.concatenate([z(2 * o1), Wm[:, 0]])
    col3 = jnp.concatenate([bs, bn, bm])
    e_mat = jnp.stack([col0, col1, col2, col3], axis=1)

    Rm = 25600
    out = _make_tc_expand(M, CD, EMB, Rm, M // Rm, 0, False)(
        xs.reshape(1, M), xa.reshape(1, M), xm.reshape(1, M),
        e_mat, Wp, bp.reshape(1, EMB))
    return out.reshape(B, T, EMB)
